# baseline (speedup 1.0000x reference)
import functools

import jax
import jax.numpy as jnp
from jax import lax
from jax.experimental import pallas as pl
from jax.experimental.pallas import tpu as pltpu
from jax.experimental.pallas import tpu_sc as plsc

N_CLS = 10000
N_CTX = 16
D = 512
SEQ = 77
SUF = 60
B = 1024

NC = 2
NS = 16
NW = NC * NS
BPW = B // NW


def _sc_body(idx_hbm, ctx_hbm, pre_hbm, suf_hbm, out_hbm,
             idx_v, combo_v, suf_v, gsem):
    wid = lax.axis_index("s") * NC + lax.axis_index("c")
    base = wid * BPW
    pltpu.sync_copy(idx_hbm.at[pl.ds(base, BPW)], idx_v)
    pltpu.sync_copy(ctx_hbm, combo_v.at[pl.ds(1, N_CTX)])

    def body(i, carry):
        b = base + i
        gp = pltpu.async_copy(pre_hbm.at[idx_v.at[i]], combo_v.at[pl.ds(0, 1)], gsem)
        gs = pltpu.async_copy(suf_hbm.at[idx_v.at[i]], suf_v, gsem)
        gp.wait()
        gs.wait()
        pltpu.sync_copy(combo_v, out_hbm.at[pl.ds(b * SEQ, 1 + N_CTX)])
        pltpu.sync_copy(suf_v.at[0], out_hbm.at[pl.ds(b * SEQ + 1 + N_CTX, SUF)])
        return carry

    lax.fori_loop(0, BPW, body, 0)


@jax.jit
def _launch(idx2, ctx, pre2, suf3):
    call = pl.kernel(
        _sc_body,
        out_type=jax.ShapeDtypeStruct((B * SEQ, D), jnp.float32),
        mesh=plsc.VectorSubcoreMesh(core_axis_name="c", subcore_axis_name="s"),
        compiler_params=pltpu.CompilerParams(use_tc_tiling_on_sc=False),
        scratch_types=[
            pltpu.VMEM((BPW, 1), jnp.int32),
            pltpu.VMEM((1 + N_CTX, D), jnp.float32),
            pltpu.VMEM((1, SUF, D), jnp.float32),
            pltpu.SemaphoreType.DMA,
        ],
    )
    return call(idx2, ctx, pre2, suf3)


def kernel(idx, ctx, token_prefix, token_suffix):
    idx2 = idx.reshape(B, 1)
    pre2 = token_prefix.reshape(N_CLS, D)
    out = _launch(idx2, ctx, pre2, token_suffix)
    return out.reshape(B, SEQ, D)


# speedup vs baseline: 3.6532x; 3.6532x over previous
import jax
import jax.numpy as jnp
from jax import lax
from jax.experimental import pallas as pl
from jax.experimental.pallas import tpu as pltpu
from jax.experimental.pallas import tpu_sc as plsc

N_CLS = 10000
N_CTX = 16
D = 512
SEQ = 77
SUF = 60
B = 1024

NC = 2
NS = 16
NW = NC * NS
BPW = B // NW
NCH = D // 16


def _row_copy(dst_ref, dst_row, src_ref, src_row):
    for c in range(NCH):
        dst_ref[0, dst_row, pl.ds(c * 16, 16)] = src_ref[0, src_row, pl.ds(c * 16, 16)]


def _sc_body(idx_hbm, ctx_hbm, pre_hbm, suf_hbm, out_hbm,
             idx_v, ctx_v, pre_v, suf_v, combo_v, gsem):
    wid = lax.axis_index("s") * NC + lax.axis_index("c")
    base = wid * BPW
    pltpu.sync_copy(idx_hbm.at[pl.ds(base, BPW)], idx_v)
    pltpu.sync_copy(ctx_hbm, ctx_v)

    def place_ctx(r, carry):
        for c in range(NCH):
            combo_v[0, 1 + r, pl.ds(c * 16, 16)] = ctx_v[r, pl.ds(c * 16, 16)]
        return carry

    lax.fori_loop(0, N_CTX, place_ctx, 0)

    vec0 = idx_v[pl.ds(0, 16)]
    vec1 = idx_v[pl.ds(16, 16)]
    lanes = lax.iota(jnp.int32, 16)

    def body(i, carry):
        v0, v1 = carry
        sel = jnp.where(i < 16, v0, v1)
        s = jnp.sum(jnp.where(lanes == (i % 16), sel, 0))
        gp = pltpu.async_copy(pre_hbm.at[pl.ds(s, 1)], pre_v, gsem)
        gs = pltpu.async_copy(suf_hbm.at[pl.ds(s, 1)], suf_v, gsem)
        gp.wait()
        gs.wait()
        _row_copy(combo_v, 0, pre_v, 0)

        def place_suf(r, c2):
            _row_copy(combo_v, 1 + N_CTX + r, suf_v, r)
            return c2

        lax.fori_loop(0, SUF, place_suf, 0)
        pltpu.sync_copy(combo_v, out_hbm.at[pl.ds(base + i, 1)])
        return carry

    lax.fori_loop(0, BPW, body, (vec0, vec1))


@jax.jit
def _launch(idx, ctx, token_prefix, token_suffix):
    call = pl.kernel(
        _sc_body,
        out_type=jax.ShapeDtypeStruct((B, SEQ, D), jnp.float32),
        mesh=plsc.VectorSubcoreMesh(core_axis_name="c", subcore_axis_name="s"),
        compiler_params=pltpu.CompilerParams(needs_layout_passes=False),
        scratch_types=[
            pltpu.VMEM((B // NW,), jnp.int32),
            pltpu.VMEM((N_CTX, D), jnp.float32),
            pltpu.VMEM((1, 1, D), jnp.float32),
            pltpu.VMEM((1, SUF, D), jnp.float32),
            pltpu.VMEM((1, SEQ, D), jnp.float32),
            pltpu.SemaphoreType.DMA,
        ],
    )
    return call(idx, ctx, token_prefix, token_suffix)


def kernel(idx, ctx, token_prefix, token_suffix):
    return _launch(idx, ctx, token_prefix, token_suffix)
